# baseline (speedup 1.0000x reference)
import functools

import jax
import jax.numpy as jnp
from jax import lax
from jax.experimental import pallas as pl
from jax.experimental.pallas import tpu as pltpu
from jax.experimental.pallas import tpu_sc as plsc

N = 10000
K = 32
D_IN = 128
D_OUT = 128

BN = 400
R_TC = 6400
R_SC = N - R_TC

SC_NC = 2
SC_NS = 16
SC_NW = SC_NC * SC_NS
CH = 8
NCH_G = R_SC // CH
LANES = 16
HREG = D_IN // LANES


def _head_body(self_ref, neigh_ref, wts_ref, wtn_ref, b_ref, out_ref):
    neigh_mean = jnp.sum(neigh_ref[...], axis=1) * (1.0 / K)
    t = jnp.dot(self_ref[...], wts_ref[...], preferred_element_type=jnp.float32)
    t = t + jnp.dot(neigh_mean, wtn_ref[...], preferred_element_type=jnp.float32)
    t = t + b_ref[...]
    c = jnp.maximum(t, 0.0)
    norm2 = jnp.sum(c * c, axis=1, keepdims=True)
    out_ref[...] = c * jax.lax.rsqrt(jnp.maximum(norm2, 1e-24))


def _tail_body(self_ref, scsum_ref, wts_ref, wtn_ref, b_ref, out_ref):
    neigh_mean = scsum_ref[...] * (1.0 / K)
    t = jnp.dot(self_ref[...], wts_ref[...], preferred_element_type=jnp.float32)
    t = t + jnp.dot(neigh_mean, wtn_ref[...], preferred_element_type=jnp.float32)
    t = t + b_ref[...]
    c = jnp.maximum(t, 0.0)
    norm2 = jnp.sum(c * c, axis=1, keepdims=True)
    out_ref[...] = c * jax.lax.rsqrt(jnp.maximum(norm2, 1e-24))


NJ = -(-NCH_G // SC_NW)
if NJ % 2:
    NJ += 1
NPAIR = NJ // 2


def _sc_body(neigh_hbm, out_hbm, buf, obuf, sem_in0, sem_in1, sem_out):
    ci = lax.axis_index("c")
    si = lax.axis_index("s")
    w = si * SC_NC + ci

    def chunk_of(j):
        return jnp.minimum(w + j * SC_NW, NCH_G - 1)

    def issue_in(j, slot, sem):
        row = R_TC + chunk_of(j) * CH
        pltpu.async_copy(neigh_hbm.at[pl.ds(row, CH)], buf.at[slot], sem)

    def wait_in(slot, sem):
        pltpu.make_async_copy(
            neigh_hbm.at[pl.ds(0, CH)], buf.at[slot], sem
        ).wait()

    issue_in(0, 0, sem_in0)
    issue_in(1, 1, sem_in1)

    def pair(p, carry):
        for b in range(2):
            j = 2 * p + b
            sem = sem_in0 if b == 0 else sem_in1
            wait_in(b, sem)

            def ibody(i, carry):
                accs = [
                    buf[b, i, 0, pl.ds(h * LANES, LANES)] for h in range(HREG)
                ]
                for kidx in range(1, K):
                    for h in range(HREG):
                        accs[h] = accs[h] + buf[b, i, kidx, pl.ds(h * LANES, LANES)]
                for h in range(HREG):
                    obuf[j, i, pl.ds(h * LANES, LANES)] = accs[h]
                return carry

            lax.fori_loop(0, CH, ibody, 0)
            pltpu.async_copy(
                obuf.at[j], out_hbm.at[pl.ds(chunk_of(j) * CH, CH)], sem_out
            )

            @pl.when(p < NPAIR - 1)
            def _():
                issue_in(j + 2, b, sem)
        return carry

    lax.fori_loop(0, NPAIR, pair, 0)

    def drain(j, carry):
        pltpu.make_async_copy(
            obuf.at[0], out_hbm.at[pl.ds(0, CH)], sem_out
        ).wait()
        return carry

    lax.fori_loop(0, NJ, drain, 0)


_sc_neigh_sum = functools.partial(
    pl.kernel,
    out_type=jax.ShapeDtypeStruct((R_SC, D_IN), jnp.float32),
    mesh=plsc.VectorSubcoreMesh(
        core_axis_name="c", subcore_axis_name="s",
        num_cores=SC_NC, num_subcores=SC_NS,
    ),
    scratch_types=[
        pltpu.VMEM((2, CH, K, D_IN), jnp.float32),
        pltpu.VMEM((NJ, CH, D_IN), jnp.float32),
        pltpu.SemaphoreType.DMA,
        pltpu.SemaphoreType.DMA,
        pltpu.SemaphoreType.DMA,
    ],
)(_sc_body)


def kernel(self_embs, neigh_embs, W_self, b_self, W_neigh, b_neigh):
    wts = W_self.T
    wtn = W_neigh.T
    b = (b_self + b_neigh).reshape(1, D_OUT)
    sc_sum = _sc_neigh_sum(neigh_embs)
    b = b + 0.0 * sc_sum[:1]

    head = pl.pallas_call(
        _head_body,
        grid=(R_TC // BN,),
        in_specs=[
            pl.BlockSpec((BN, D_IN), lambda i: (i, 0)),
            pl.BlockSpec((BN, K, D_IN), lambda i: (i, 0, 0)),
            pl.BlockSpec((D_IN, D_OUT), lambda i: (0, 0)),
            pl.BlockSpec((D_IN, D_OUT), lambda i: (0, 0)),
            pl.BlockSpec((1, D_OUT), lambda i: (0, 0)),
        ],
        out_specs=pl.BlockSpec((BN, D_OUT), lambda i: (i, 0)),
        out_shape=jax.ShapeDtypeStruct((R_TC, D_OUT), jnp.float32),
        compiler_params=pltpu.CompilerParams(
            dimension_semantics=("arbitrary",),
        ),
    )(self_embs, neigh_embs, wts, wtn, b)

    nhead = R_TC // BN
    tail = pl.pallas_call(
        _tail_body,
        grid=(R_SC // BN,),
        in_specs=[
            pl.BlockSpec((BN, D_IN), lambda i: (i + nhead, 0)),
            pl.BlockSpec((BN, D_IN), lambda i: (i, 0)),
            pl.BlockSpec((D_IN, D_OUT), lambda i: (0, 0)),
            pl.BlockSpec((D_IN, D_OUT), lambda i: (0, 0)),
            pl.BlockSpec((1, D_OUT), lambda i: (0, 0)),
        ],
        out_specs=pl.BlockSpec((BN, D_OUT), lambda i: (i, 0)),
        out_shape=jax.ShapeDtypeStruct((R_SC, D_OUT), jnp.float32),
        compiler_params=pltpu.CompilerParams(
            dimension_semantics=("arbitrary",),
        ),
    )(self_embs, sc_sum, wts, wtn, b)

    return jnp.concatenate([head, tail], axis=0)


# speedup vs baseline: 1.1759x; 1.1759x over previous
import functools

import jax
import jax.numpy as jnp
from jax import lax
from jax.experimental import pallas as pl
from jax.experimental.pallas import tpu as pltpu
from jax.experimental.pallas import tpu_sc as plsc

N = 10000
K = 32
D_IN = 128
D_OUT = 128

BN = 400
R_TC = 6400
R_SC = N - R_TC

SC_NC = 2
SC_NS = 16
SC_NW = SC_NC * SC_NS
CH = 8
NCH_G = R_SC // CH
LANES = 16
HREG = D_IN // LANES


def _head_body(self_ref, neigh_ref, wts_ref, wtn_ref, b_ref, out_ref):
    neigh_mean = jnp.sum(neigh_ref[...], axis=1) * (1.0 / K)
    t = jnp.dot(self_ref[...], wts_ref[...], preferred_element_type=jnp.float32)
    t = t + jnp.dot(neigh_mean, wtn_ref[...], preferred_element_type=jnp.float32)
    t = t + b_ref[...]
    c = jnp.maximum(t, 0.0)
    norm2 = jnp.sum(c * c, axis=1, keepdims=True)
    out_ref[...] = c * jax.lax.rsqrt(jnp.maximum(norm2, 1e-24))


def _tail_body(self_ref, scsum_ref, wts_ref, wtn_ref, b_ref, out_ref):
    neigh_mean = scsum_ref[...] * (1.0 / K)
    t = jnp.dot(self_ref[...], wts_ref[...], preferred_element_type=jnp.float32)
    t = t + jnp.dot(neigh_mean, wtn_ref[...], preferred_element_type=jnp.float32)
    t = t + b_ref[...]
    c = jnp.maximum(t, 0.0)
    norm2 = jnp.sum(c * c, axis=1, keepdims=True)
    out_ref[...] = c * jax.lax.rsqrt(jnp.maximum(norm2, 1e-24))


NJ = -(-NCH_G // SC_NW)
if NJ % 2:
    NJ += 1
NPAIR = NJ // 2


def _sc_body(neigh_hbm, out_hbm, buf, obuf, sem_in0, sem_in1, sem_out):
    ci = lax.axis_index("c")
    si = lax.axis_index("s")
    w = si * SC_NC + ci

    def chunk_of(j):
        return jnp.minimum(w + j * SC_NW, NCH_G - 1)

    def issue_in(j, slot, sem):
        row = R_TC + chunk_of(j) * CH
        pltpu.async_copy(neigh_hbm.at[pl.ds(row, CH)], buf.at[slot], sem)

    def wait_in(slot, sem):
        pltpu.make_async_copy(
            neigh_hbm.at[pl.ds(0, CH)], buf.at[slot], sem
        ).wait()

    issue_in(0, 0, sem_in0)
    issue_in(1, 1, sem_in1)

    def pair(p, carry):
        for b in range(2):
            j = 2 * p + b
            sem = sem_in0 if b == 0 else sem_in1
            wait_in(b, sem)

            def ibody(i, carry):
                accs = [
                    buf[b, i, 0, pl.ds(h * LANES, LANES)] for h in range(HREG)
                ]
                for kidx in range(1, K):
                    for h in range(HREG):
                        accs[h] = accs[h] + buf[b, i, kidx, pl.ds(h * LANES, LANES)]
                for h in range(HREG):
                    obuf[j, i, pl.ds(h * LANES, LANES)] = accs[h]
                return carry

            lax.fori_loop(0, CH, ibody, 0)
            pltpu.async_copy(
                obuf.at[j], out_hbm.at[pl.ds(chunk_of(j) * CH, CH)], sem_out
            )

            @pl.when(p < NPAIR - 1)
            def _():
                issue_in(j + 2, b, sem)
        return carry

    lax.fori_loop(0, NPAIR, pair, 0)

    def drain(j, carry):
        pltpu.make_async_copy(
            obuf.at[0], out_hbm.at[pl.ds(0, CH)], sem_out
        ).wait()
        return carry

    lax.fori_loop(0, NJ, drain, 0)


_sc_neigh_sum = functools.partial(
    pl.kernel,
    out_type=jax.ShapeDtypeStruct((R_SC, D_IN), jnp.float32),
    mesh=plsc.VectorSubcoreMesh(
        core_axis_name="c", subcore_axis_name="s",
        num_cores=SC_NC, num_subcores=SC_NS,
    ),
    scratch_types=[
        pltpu.VMEM((2, CH, K, D_IN), jnp.float32),
        pltpu.VMEM((NJ, CH, D_IN), jnp.float32),
        pltpu.SemaphoreType.DMA,
        pltpu.SemaphoreType.DMA,
        pltpu.SemaphoreType.DMA,
    ],
)(_sc_body)


def kernel(self_embs, neigh_embs, W_self, b_self, W_neigh, b_neigh):
    wts = W_self.T
    wtn = W_neigh.T
    b = (b_self + b_neigh).reshape(1, D_OUT)
    sc_sum = _sc_neigh_sum(neigh_embs)

    head = pl.pallas_call(
        _head_body,
        grid=(R_TC // BN,),
        in_specs=[
            pl.BlockSpec((BN, D_IN), lambda i: (i, 0)),
            pl.BlockSpec((BN, K, D_IN), lambda i: (i, 0, 0)),
            pl.BlockSpec((D_IN, D_OUT), lambda i: (0, 0)),
            pl.BlockSpec((D_IN, D_OUT), lambda i: (0, 0)),
            pl.BlockSpec((1, D_OUT), lambda i: (0, 0)),
        ],
        out_specs=pl.BlockSpec((BN, D_OUT), lambda i: (i, 0)),
        out_shape=jax.ShapeDtypeStruct((R_TC, D_OUT), jnp.float32),
        compiler_params=pltpu.CompilerParams(
            dimension_semantics=("arbitrary",),
        ),
    )(self_embs, neigh_embs, wts, wtn, b)

    nhead = R_TC // BN
    tail = pl.pallas_call(
        _tail_body,
        grid=(R_SC // BN,),
        in_specs=[
            pl.BlockSpec((BN, D_IN), lambda i: (i + nhead, 0)),
            pl.BlockSpec((BN, D_IN), lambda i: (i, 0)),
            pl.BlockSpec((D_IN, D_OUT), lambda i: (0, 0)),
            pl.BlockSpec((D_IN, D_OUT), lambda i: (0, 0)),
            pl.BlockSpec((1, D_OUT), lambda i: (0, 0)),
        ],
        out_specs=pl.BlockSpec((BN, D_OUT), lambda i: (i, 0)),
        out_shape=jax.ShapeDtypeStruct((R_SC, D_OUT), jnp.float32),
        compiler_params=pltpu.CompilerParams(
            dimension_semantics=("arbitrary",),
        ),
    )(self_embs, sc_sum, wts, wtn, b)

    return jnp.concatenate([head, tail], axis=0)


# speedup vs baseline: 1.2542x; 1.0666x over previous
import functools

import jax
import jax.numpy as jnp
from jax import lax
from jax.experimental import pallas as pl
from jax.experimental.pallas import tpu as pltpu
from jax.experimental.pallas import tpu_sc as plsc

N = 10000
K = 32
D_IN = 128
D_OUT = 128

BN = 400
R_TC = 8400
R_SC = N - R_TC

SC_NC = 2
SC_NS = 16
SC_NW = SC_NC * SC_NS
CH = 8
NCH_G = R_SC // CH
LANES = 16
HREG = D_IN // LANES


def _head_body(self_ref, neigh_ref, wts_ref, wtn_ref, b_ref, out_ref):
    neigh_mean = jnp.sum(neigh_ref[...], axis=1) * (1.0 / K)
    t = jnp.dot(self_ref[...], wts_ref[...], preferred_element_type=jnp.float32)
    t = t + jnp.dot(neigh_mean, wtn_ref[...], preferred_element_type=jnp.float32)
    t = t + b_ref[...]
    c = jnp.maximum(t, 0.0)
    norm2 = jnp.sum(c * c, axis=1, keepdims=True)
    out_ref[...] = c * jax.lax.rsqrt(jnp.maximum(norm2, 1e-24))


def _tail_body(self_ref, scsum_ref, wts_ref, wtn_ref, b_ref, out_ref):
    neigh_mean = scsum_ref[...] * (1.0 / K)
    t = jnp.dot(self_ref[...], wts_ref[...], preferred_element_type=jnp.float32)
    t = t + jnp.dot(neigh_mean, wtn_ref[...], preferred_element_type=jnp.float32)
    t = t + b_ref[...]
    c = jnp.maximum(t, 0.0)
    norm2 = jnp.sum(c * c, axis=1, keepdims=True)
    out_ref[...] = c * jax.lax.rsqrt(jnp.maximum(norm2, 1e-24))


NJ = -(-NCH_G // SC_NW)
if NJ % 2:
    NJ += 1
NPAIR = NJ // 2


def _sc_body(neigh_hbm, out_hbm, buf, obuf, sem_in0, sem_in1, sem_out):
    ci = lax.axis_index("c")
    si = lax.axis_index("s")
    w = si * SC_NC + ci

    def chunk_of(j):
        return jnp.minimum(w + j * SC_NW, NCH_G - 1)

    def issue_in(j, slot, sem):
        row = R_TC + chunk_of(j) * CH
        pltpu.async_copy(neigh_hbm.at[pl.ds(row, CH)], buf.at[slot], sem)

    def wait_in(slot, sem):
        pltpu.make_async_copy(
            neigh_hbm.at[pl.ds(0, CH)], buf.at[slot], sem
        ).wait()

    issue_in(0, 0, sem_in0)
    issue_in(1, 1, sem_in1)

    def pair(p, carry):
        for b in range(2):
            j = 2 * p + b
            sem = sem_in0 if b == 0 else sem_in1
            wait_in(b, sem)

            def ibody(i, carry):
                accs = [
                    buf[b, i, 0, pl.ds(h * LANES, LANES)] for h in range(HREG)
                ]
                for kidx in range(1, K):
                    for h in range(HREG):
                        accs[h] = accs[h] + buf[b, i, kidx, pl.ds(h * LANES, LANES)]
                for h in range(HREG):
                    obuf[j, i, pl.ds(h * LANES, LANES)] = accs[h]
                return carry

            lax.fori_loop(0, CH, ibody, 0)
            pltpu.async_copy(
                obuf.at[j], out_hbm.at[pl.ds(chunk_of(j) * CH, CH)], sem_out
            )

            @pl.when(p < NPAIR - 1)
            def _():
                issue_in(j + 2, b, sem)
        return carry

    lax.fori_loop(0, NPAIR, pair, 0)

    def drain(j, carry):
        pltpu.make_async_copy(
            obuf.at[0], out_hbm.at[pl.ds(0, CH)], sem_out
        ).wait()
        return carry

    lax.fori_loop(0, NJ, drain, 0)


_sc_neigh_sum = functools.partial(
    pl.kernel,
    out_type=jax.ShapeDtypeStruct((R_SC, D_IN), jnp.float32),
    mesh=plsc.VectorSubcoreMesh(
        core_axis_name="c", subcore_axis_name="s",
        num_cores=SC_NC, num_subcores=SC_NS,
    ),
    scratch_types=[
        pltpu.VMEM((2, CH, K, D_IN), jnp.float32),
        pltpu.VMEM((NJ, CH, D_IN), jnp.float32),
        pltpu.SemaphoreType.DMA,
        pltpu.SemaphoreType.DMA,
        pltpu.SemaphoreType.DMA,
    ],
)(_sc_body)


def kernel(self_embs, neigh_embs, W_self, b_self, W_neigh, b_neigh):
    wts = W_self.T
    wtn = W_neigh.T
    b = (b_self + b_neigh).reshape(1, D_OUT)
    sc_sum = _sc_neigh_sum(neigh_embs)

    head = pl.pallas_call(
        _head_body,
        grid=(R_TC // BN,),
        in_specs=[
            pl.BlockSpec((BN, D_IN), lambda i: (i, 0)),
            pl.BlockSpec((BN, K, D_IN), lambda i: (i, 0, 0)),
            pl.BlockSpec((D_IN, D_OUT), lambda i: (0, 0)),
            pl.BlockSpec((D_IN, D_OUT), lambda i: (0, 0)),
            pl.BlockSpec((1, D_OUT), lambda i: (0, 0)),
        ],
        out_specs=pl.BlockSpec((BN, D_OUT), lambda i: (i, 0)),
        out_shape=jax.ShapeDtypeStruct((R_TC, D_OUT), jnp.float32),
        compiler_params=pltpu.CompilerParams(
            dimension_semantics=("arbitrary",),
        ),
    )(self_embs, neigh_embs, wts, wtn, b)

    nhead = R_TC // BN
    tail = pl.pallas_call(
        _tail_body,
        grid=(R_SC // BN,),
        in_specs=[
            pl.BlockSpec((BN, D_IN), lambda i: (i + nhead, 0)),
            pl.BlockSpec((BN, D_IN), lambda i: (i, 0)),
            pl.BlockSpec((D_IN, D_OUT), lambda i: (0, 0)),
            pl.BlockSpec((D_IN, D_OUT), lambda i: (0, 0)),
            pl.BlockSpec((1, D_OUT), lambda i: (0, 0)),
        ],
        out_specs=pl.BlockSpec((BN, D_OUT), lambda i: (i, 0)),
        out_shape=jax.ShapeDtypeStruct((R_SC, D_OUT), jnp.float32),
        compiler_params=pltpu.CompilerParams(
            dimension_semantics=("arbitrary",),
        ),
    )(self_embs, sc_sum, wts, wtn, b)

    return jnp.concatenate([head, tail], axis=0)


# speedup vs baseline: 1.9710x; 1.5716x over previous
import jax
import jax.numpy as jnp
from jax.experimental import pallas as pl
from jax.experimental.pallas import tpu as pltpu

N = 10000
K = 32
D_IN = 128
D_OUT = 128

BN = 400


def _body(self_ref, neigh_ref, wts_ref, wtn_ref, b_ref, out_ref):
    neigh_mean = jnp.sum(neigh_ref[...], axis=1) * (1.0 / K)
    t = jnp.dot(self_ref[...], wts_ref[...], preferred_element_type=jnp.float32)
    t = t + jnp.dot(neigh_mean, wtn_ref[...], preferred_element_type=jnp.float32)
    t = t + b_ref[...]
    c = jnp.maximum(t, 0.0)
    norm2 = jnp.sum(c * c, axis=1, keepdims=True)
    out_ref[...] = c * jax.lax.rsqrt(jnp.maximum(norm2, 1e-24))


def kernel(self_embs, neigh_embs, W_self, b_self, W_neigh, b_neigh):
    wts = W_self.T
    wtn = W_neigh.T
    b = (b_self + b_neigh).reshape(1, D_OUT)
    return pl.pallas_call(
        _body,
        grid=(N // BN,),
        in_specs=[
            pl.BlockSpec((BN, D_IN), lambda i: (i, 0)),
            pl.BlockSpec((BN, K, D_IN), lambda i: (i, 0, 0)),
            pl.BlockSpec((D_IN, D_OUT), lambda i: (0, 0)),
            pl.BlockSpec((D_IN, D_OUT), lambda i: (0, 0)),
            pl.BlockSpec((1, D_OUT), lambda i: (0, 0)),
        ],
        out_specs=pl.BlockSpec((BN, D_OUT), lambda i: (i, 0)),
        out_shape=jax.ShapeDtypeStruct((N, D_OUT), jnp.float32),
        compiler_params=pltpu.CompilerParams(
            dimension_semantics=("arbitrary",),
        ),
    )(self_embs, neigh_embs, wts, wtn, b)
